# baseline (speedup 1.0000x reference)
import functools

import jax
import jax.numpy as jnp
from jax import lax
from jax.experimental import pallas as pl
from jax.experimental.pallas import tpu as pltpu
from jax.experimental.pallas import tpu_sc as plsc

B = 128
N = 4096
TOPK = 32
IOU_CUTOFF = 0.5
EPS = 1e-6
ALPHA = 0.25
GAMMA = 2.0
TIE = 2.0 ** -28
NEG = -3.0

NC = 2
NS = 16
L = 16
NW = NC * NS
RPW = B // NW
CHUNKS = N // L
NG = 16
GC = CHUNKS // NG


def _focal_terms(l, t):
    q = jnp.exp(-jnp.abs(l))
    ce = jnp.maximum(l, 0.0) - l * t + jnp.log1p(q)
    r = 1.0 / (1.0 + q)
    p = jnp.where(l >= 0.0, r, q * r)
    p_t = p * t + (1.0 - p) * (1.0 - t)
    alpha_t = ALPHA * t + (1.0 - ALPHA) * (1.0 - t)
    return alpha_t * ce * (1.0 - p_t) ** 2


def _bfmax(v):
    lane = lax.broadcasted_iota(jnp.int32, (L,), 0)
    for k in (1, 2, 4, 8):
        v = jnp.maximum(v, jnp.take(v, lane ^ k))
    return v


def _sc_body(s_hbm, e_hbm, g0_hbm, g1_hbm, out_m,
             s_v, e_v, g_v, key_v, gm_v, mask_v):
    wid = lax.axis_index("s") * NC + lax.axis_index("c")
    lane = lax.broadcasted_iota(jnp.int32, (L,), 0)

    for j in range(RPW):
        row = wid * RPW + j

        pltpu.sync_copy(s_hbm.at[pl.ds(row * N, N)], s_v)
        pltpu.sync_copy(e_hbm.at[pl.ds(row * N, N)], e_v)
        pltpu.sync_copy(g0_hbm.at[pl.ds(row * L, L)], g_v)
        g0 = g_v[...]
        pltpu.sync_copy(g1_hbm.at[pl.ds(row * L, L)], g_v)
        g1 = g_v[...]

        def grp(g, _):
            def chunk(k, acc):
                i = g * GC + k
                s = s_v[pl.ds(i * L, L)]
                e = e_v[pl.ds(i * L, L)]
                inter = jnp.maximum(
                    jnp.minimum(e, g1) - jnp.maximum(s, g0), 0.0)
                union = (e - s) + (g1 - g0) - inter
                enclose = jnp.maximum(e, g1) - jnp.minimum(s, g0)
                giou = (inter / (union + EPS)
                        - (enclose - union) / (enclose + EPS))
                key = giou - (lane + i * L).astype(jnp.float32) * TIE
                key_v[pl.ds(i * L, L)] = key
                return jnp.maximum(acc, key)

            acc = lax.fori_loop(0, GC, chunk, jnp.full((L,), NEG))
            gm_v[pl.ds(g * L, L)] = acc
            return _

        lax.fori_loop(0, NG, grp, 0)

        def ext(_, m_prev):
            def fmax(g, acc):
                return jnp.maximum(acc, gm_v[pl.ds(g * L, L)])

            mall = lax.fori_loop(0, NG, fmax, jnp.full((L,), NEG))
            m = _bfmax(mall)

            def floc(g, acc):
                gv = gm_v[pl.ds(g * L, L)]
                return jnp.where(gv == m, jnp.float32(g), acc)

            accv = lax.fori_loop(0, NG, floc, jnp.full((L,), -1.0))
            gs = _bfmax(accv)[0].astype(jnp.int32)

            def rescan(k, acc):
                v = key_v[pl.ds(gs * (GC * L) + k * L, L)]
                return jnp.maximum(acc, jnp.where(v >= m, NEG, v))

            nacc = lax.fori_loop(0, GC, rescan, jnp.full((L,), NEG))
            gm_v[pl.ds(gs * L, L)] = nacc
            return m

        m32 = lax.fori_loop(0, TOPK, ext, jnp.full((L,), 999.0))

        def mk(i, _):
            kv = key_v[pl.ds(i * L, L)]
            mask_v[pl.ds(i * L, L)] = jnp.where(kv >= m32, 1.0, 0.0)
            return _

        lax.fori_loop(0, CHUNKS, mk, 0)
        pltpu.sync_copy(mask_v, out_m.at[pl.ds(row * N, N)])


def _sc_topk(s, e, g0rep, g1rep):
    mesh = plsc.VectorSubcoreMesh(core_axis_name="c", subcore_axis_name="s")
    f = pl.kernel(
        _sc_body,
        mesh=mesh,
        out_type=[jax.ShapeDtypeStruct((B * N,), jnp.float32)],
        scratch_types=[
            pltpu.VMEM((N,), jnp.float32),
            pltpu.VMEM((N,), jnp.float32),
            pltpu.VMEM((L,), jnp.float32),
            pltpu.VMEM((N,), jnp.float32),
            pltpu.VMEM((NG * L,), jnp.float32),
            pltpu.VMEM((N,), jnp.float32),
        ],
    )
    return f(s.reshape(-1), e.reshape(-1),
             g0rep.reshape(-1), g1rep.reshape(-1))


def _b1_body(s_ref, e_ref, sc_ref, gt_ref, base_ref, delta_ref):
    s = s_ref[:, :]
    e = e_ref[:, :]
    g0 = gt_ref[:, 0:1]
    g1 = gt_ref[:, 1:2]

    inter = jnp.clip(jnp.minimum(e, g1) - jnp.maximum(s, g0), 0.0)
    union = (e - s) + (g1 - g0) - inter
    enclose = jnp.maximum(e, g1) - jnp.minimum(s, g0)
    giou = inter / (union + EPS) - (enclose - union) / (enclose + EPS)

    t = jnp.where(giou < IOU_CUTOFF, 0.0, giou)
    l = sc_ref[:, :]
    ft = _focal_terms(l, t)
    base_ref[0, 0] = jnp.sum(ft)
    delta_ref[:, :] = _focal_terms(l, 1.0) - ft


def _b2_body(m_ref, d_ref, s_ref, e_ref, gt_ref, base_ref, out_ref):
    m = m_ref[:, :]
    g0 = gt_ref[:, 0:1]
    g1 = gt_ref[:, 1:2]
    corr = jnp.sum(m * d_ref[:, :])
    l1 = jnp.abs(s_ref[:, :] - g0) + jnp.abs(e_ref[:, :] - g1)
    l1_sum = jnp.sum(m * l1)
    out_ref[0, 0] = ((base_ref[0, 0] + corr) / (B * N)
                     + l1_sum / (B * TOPK * 2))


def kernel(proposal, score, gt):
    s = proposal[:, :, 0]
    e = proposal[:, :, 1]
    g0rep = jnp.broadcast_to(gt[:, 0:1], (B, L))
    g1rep = jnp.broadcast_to(gt[:, 1:2], (B, L))

    (mask,) = _sc_topk(s, e, g0rep, g1rep)

    base, delta = pl.pallas_call(
        _b1_body,
        out_shape=[
            jax.ShapeDtypeStruct((1, 1), jnp.float32),
            jax.ShapeDtypeStruct((B, N), jnp.float32),
        ],
        out_specs=[
            pl.BlockSpec(memory_space=pltpu.SMEM),
            pl.BlockSpec(memory_space=pltpu.VMEM),
        ],
    )(s, e, score, gt)

    out = pl.pallas_call(
        _b2_body,
        out_shape=jax.ShapeDtypeStruct((1, 1), jnp.float32),
        out_specs=pl.BlockSpec(memory_space=pltpu.SMEM),
    )(mask.reshape(B, N), delta, s, e, gt, base)
    return out[0, 0]


# speedup vs baseline: 1.3132x; 1.3132x over previous
import functools

import jax
import jax.numpy as jnp
from jax import lax
from jax.experimental import pallas as pl
from jax.experimental.pallas import tpu as pltpu
from jax.experimental.pallas import tpu_sc as plsc

B = 128
N = 4096
TOPK = 32
IOU_CUTOFF = 0.5
EPS = 1e-6
ALPHA = 0.25
GAMMA = 2.0
TIE = 2.0 ** -28
NEG = -3.0

NC = 2
NS = 16
L = 16
NW = NC * NS
RPW = B // NW
CHUNKS = N // L
NG = 16
GC = CHUNKS // NG


def _focal_terms(l, t):
    q = jnp.exp(-jnp.abs(l))
    ce = jnp.maximum(l, 0.0) - l * t + jnp.log1p(q)
    r = 1.0 / (1.0 + q)
    p = jnp.where(l >= 0.0, r, q * r)
    p_t = p * t + (1.0 - p) * (1.0 - t)
    alpha_t = ALPHA * t + (1.0 - ALPHA) * (1.0 - t)
    return alpha_t * ce * (1.0 - p_t) ** 2


def _bfmax(v):
    lane = lax.broadcasted_iota(jnp.int32, (L,), 0)
    for k in (1, 2, 4, 8):
        v = jnp.maximum(v, jnp.take(v, lane ^ k))
    return v


def _sc_body(s_hbm, e_hbm, g0_hbm, g1_hbm, out_m,
             s_v, e_v, g0_v, g1_v, key_v, gm_v, mask_v):
    wid = lax.axis_index("s") * NC + lax.axis_index("c")
    lane = lax.broadcasted_iota(jnp.int32, (L,), 0)
    r0 = wid * RPW

    pltpu.sync_copy(s_hbm.at[pl.ds(r0 * N, RPW * N)], s_v)
    pltpu.sync_copy(e_hbm.at[pl.ds(r0 * N, RPW * N)], e_v)
    pltpu.sync_copy(g0_hbm.at[pl.ds(r0 * L, RPW * L)], g0_v)
    pltpu.sync_copy(g1_hbm.at[pl.ds(r0 * L, RPW * L)], g1_v)
    g0s = [g0_v[pl.ds(r * L, L)] for r in range(RPW)]
    g1s = [g1_v[pl.ds(r * L, L)] for r in range(RPW)]

    neg = jnp.full((L,), NEG)

    def grp(g, _):
        def chunk(k, accs):
            i = g * GC + k
            colf = (lane + i * L).astype(jnp.float32) * TIE
            new = []
            for r in range(RPW):
                off = r * N + i * L
                s = s_v[pl.ds(off, L)]
                e = e_v[pl.ds(off, L)]
                inter = jnp.maximum(
                    jnp.minimum(e, g1s[r]) - jnp.maximum(s, g0s[r]), 0.0)
                union = (e - s) + (g1s[r] - g0s[r]) - inter
                enclose = jnp.maximum(e, g1s[r]) - jnp.minimum(s, g0s[r])
                giou = (inter / (union + EPS)
                        - (enclose - union) / (enclose + EPS))
                key = giou - colf
                key_v[pl.ds(off, L)] = key
                new.append(jnp.maximum(accs[r], key))
            return tuple(new)

        accs = lax.fori_loop(0, GC, chunk, (neg,) * RPW)
        for r in range(RPW):
            gm_v[pl.ds((r * NG + g) * L, L)] = accs[r]
        return _

    lax.fori_loop(0, NG, grp, 0)

    def ext(step, m32s):
        out = []
        for r in range(RPW):
            base = r * NG * L
            gl = [gm_v[pl.ds(base + g * L, L)] for g in range(NG)]
            mall = gl[0]
            for g in range(1, NG):
                mall = jnp.maximum(mall, gl[g])
            m = _bfmax(mall)
            accv = jnp.full((L,), -1.0)
            for g in range(NG):
                accv = jnp.where(gl[g] == m, jnp.float32(g), accv)
            gs = _bfmax(accv)[0].astype(jnp.int32)
            nacc = neg
            rb = r * N + gs * (GC * L)
            for k in range(GC):
                v = key_v[pl.ds(rb + k * L, L)]
                nacc = jnp.maximum(nacc, jnp.where(v >= m, NEG, v))
            gm_v[pl.ds(base + gs * L, L)] = nacc
            out.append(m)
        return tuple(out)

    m32s = lax.fori_loop(0, TOPK, ext, (jnp.full((L,), 999.0),) * RPW)

    def mk(i, _):
        for r in range(RPW):
            off = r * N + i * L
            kv = key_v[pl.ds(off, L)]
            mask_v[pl.ds(off, L)] = jnp.where(kv >= m32s[r], 1.0, 0.0)
        return _

    lax.fori_loop(0, CHUNKS, mk, 0)
    pltpu.sync_copy(mask_v, out_m.at[pl.ds(r0 * N, RPW * N)])


def _sc_topk(s, e, g0rep, g1rep):
    mesh = plsc.VectorSubcoreMesh(core_axis_name="c", subcore_axis_name="s")
    f = pl.kernel(
        _sc_body,
        mesh=mesh,
        out_type=[jax.ShapeDtypeStruct((B * N,), jnp.float32)],
        scratch_types=[
            pltpu.VMEM((RPW * N,), jnp.float32),
            pltpu.VMEM((RPW * N,), jnp.float32),
            pltpu.VMEM((RPW * L,), jnp.float32),
            pltpu.VMEM((RPW * L,), jnp.float32),
            pltpu.VMEM((RPW * N,), jnp.float32),
            pltpu.VMEM((RPW * NG * L,), jnp.float32),
            pltpu.VMEM((RPW * N,), jnp.float32),
        ],
    )
    return f(s.reshape(-1), e.reshape(-1),
             g0rep.reshape(-1), g1rep.reshape(-1))


def _b1_body(s_ref, e_ref, sc_ref, gt_ref, base_ref, delta_ref):
    s = s_ref[:, :]
    e = e_ref[:, :]
    g0 = gt_ref[:, 0:1]
    g1 = gt_ref[:, 1:2]

    inter = jnp.clip(jnp.minimum(e, g1) - jnp.maximum(s, g0), 0.0)
    union = (e - s) + (g1 - g0) - inter
    enclose = jnp.maximum(e, g1) - jnp.minimum(s, g0)
    giou = inter / (union + EPS) - (enclose - union) / (enclose + EPS)

    t = jnp.where(giou < IOU_CUTOFF, 0.0, giou)
    l = sc_ref[:, :]
    ft = _focal_terms(l, t)
    base_ref[0, 0] = jnp.sum(ft)
    delta_ref[:, :] = _focal_terms(l, 1.0) - ft


def _b2_body(m_ref, d_ref, s_ref, e_ref, gt_ref, base_ref, out_ref):
    m = m_ref[:, :]
    g0 = gt_ref[:, 0:1]
    g1 = gt_ref[:, 1:2]
    corr = jnp.sum(m * d_ref[:, :])
    l1 = jnp.abs(s_ref[:, :] - g0) + jnp.abs(e_ref[:, :] - g1)
    l1_sum = jnp.sum(m * l1)
    out_ref[0, 0] = ((base_ref[0, 0] + corr) / (B * N)
                     + l1_sum / (B * TOPK * 2))


def kernel(proposal, score, gt):
    s = proposal[:, :, 0]
    e = proposal[:, :, 1]
    g0rep = jnp.broadcast_to(gt[:, 0:1], (B, L))
    g1rep = jnp.broadcast_to(gt[:, 1:2], (B, L))

    (mask,) = _sc_topk(s, e, g0rep, g1rep)

    base, delta = pl.pallas_call(
        _b1_body,
        out_shape=[
            jax.ShapeDtypeStruct((1, 1), jnp.float32),
            jax.ShapeDtypeStruct((B, N), jnp.float32),
        ],
        out_specs=[
            pl.BlockSpec(memory_space=pltpu.SMEM),
            pl.BlockSpec(memory_space=pltpu.VMEM),
        ],
    )(s, e, score, gt)

    out = pl.pallas_call(
        _b2_body,
        out_shape=jax.ShapeDtypeStruct((1, 1), jnp.float32),
        out_specs=pl.BlockSpec(memory_space=pltpu.SMEM),
    )(mask.reshape(B, N), delta, s, e, gt, base)
    return out[0, 0]


# speedup vs baseline: 1.3178x; 1.0035x over previous
import functools

import jax
import jax.numpy as jnp
from jax import lax
from jax.experimental import pallas as pl
from jax.experimental.pallas import tpu as pltpu
from jax.experimental.pallas import tpu_sc as plsc

B = 128
N = 4096
TOPK = 32
IOU_CUTOFF = 0.5
EPS = 1e-6
ALPHA = 0.25
GAMMA = 2.0
TIE = 2.0 ** -28
NEG = -3.0

NC = 2
NS = 16
L = 16
NW = NC * NS
RPW = B // NW
CHUNKS = N // L
NG = 16
GC = CHUNKS // NG


def _focal_terms(l, t):
    q = jnp.exp(-jnp.abs(l))
    ce = jnp.maximum(l, 0.0) - l * t + jnp.log1p(q)
    r = 1.0 / (1.0 + q)
    p = jnp.where(l >= 0.0, r, q * r)
    p_t = p * t + (1.0 - p) * (1.0 - t)
    alpha_t = ALPHA * t + (1.0 - ALPHA) * (1.0 - t)
    return alpha_t * ce * (1.0 - p_t) ** 2


def _bfmax(v):
    lane = lax.broadcasted_iota(jnp.int32, (L,), 0)
    for k in (1, 2, 4, 8):
        v = jnp.maximum(v, jnp.take(v, lane ^ k))
    return v


def _sc_body(s_hbm, e_hbm, g0_hbm, g1_hbm, out_m, out_l1,
             s_v, e_v, g0_v, g1_v, key_v, gm_v, mask_v, l1_v):
    wid = lax.axis_index("s") * NC + lax.axis_index("c")
    lane = lax.broadcasted_iota(jnp.int32, (L,), 0)
    r0 = wid * RPW

    pltpu.sync_copy(s_hbm.at[pl.ds(r0 * N, RPW * N)], s_v)
    pltpu.sync_copy(e_hbm.at[pl.ds(r0 * N, RPW * N)], e_v)
    pltpu.sync_copy(g0_hbm.at[pl.ds(r0 * L, RPW * L)], g0_v)
    pltpu.sync_copy(g1_hbm.at[pl.ds(r0 * L, RPW * L)], g1_v)
    g0s = [g0_v[pl.ds(r * L, L)] for r in range(RPW)]
    g1s = [g1_v[pl.ds(r * L, L)] for r in range(RPW)]

    neg = jnp.full((L,), NEG)

    def grp(g, _):
        def chunk(k, accs):
            i = g * GC + k
            colf = (lane + i * L).astype(jnp.float32) * TIE
            new = []
            for r in range(RPW):
                off = r * N + i * L
                s = s_v[pl.ds(off, L)]
                e = e_v[pl.ds(off, L)]
                inter = jnp.maximum(
                    jnp.minimum(e, g1s[r]) - jnp.maximum(s, g0s[r]), 0.0)
                union = (e - s) + (g1s[r] - g0s[r]) - inter
                enclose = jnp.maximum(e, g1s[r]) - jnp.minimum(s, g0s[r])
                giou = (inter / (union + EPS)
                        - (enclose - union) / (enclose + EPS))
                key = giou - colf
                key_v[pl.ds(off, L)] = key
                new.append(jnp.maximum(accs[r], key))
            return tuple(new)

        accs = lax.fori_loop(0, GC, chunk, (neg,) * RPW)
        for r in range(RPW):
            gm_v[pl.ds((r * NG + g) * L, L)] = accs[r]
        return _

    lax.fori_loop(0, NG, grp, 0)

    def ext(step, m32s):
        out = []
        for r in range(RPW):
            base = r * NG * L
            gl = [gm_v[pl.ds(base + g * L, L)] for g in range(NG)]
            mall = gl[0]
            for g in range(1, NG):
                mall = jnp.maximum(mall, gl[g])
            m = _bfmax(mall)
            accv = jnp.full((L,), -1.0)
            for g in range(NG):
                accv = jnp.where(gl[g] == m, jnp.float32(g), accv)
            gs = _bfmax(accv)[0].astype(jnp.int32)
            nacc = neg
            rb = r * N + gs * (GC * L)
            for k in range(GC):
                v = key_v[pl.ds(rb + k * L, L)]
                nacc = jnp.maximum(nacc, jnp.where(v >= m, NEG, v))
            gm_v[pl.ds(base + gs * L, L)] = nacc
            out.append(m)
        return tuple(out)

    m32s = lax.fori_loop(0, TOPK, ext, (jnp.full((L,), 999.0),) * RPW)

    def mk(i, accs):
        new = []
        for r in range(RPW):
            off = r * N + i * L
            kv = key_v[pl.ds(off, L)]
            mv = jnp.where(kv >= m32s[r], 1.0, 0.0)
            mask_v[pl.ds(off, L)] = mv
            s = s_v[pl.ds(off, L)]
            e = e_v[pl.ds(off, L)]
            l1 = jnp.abs(s - g0s[r]) + jnp.abs(e - g1s[r])
            new.append(accs[r] + mv * l1)
        return tuple(new)

    l1accs = lax.fori_loop(0, CHUNKS, mk, (jnp.zeros((L,)),) * RPW)
    for r in range(RPW):
        l1_v[pl.ds(r * L, L)] = l1accs[r]
    pltpu.sync_copy(mask_v, out_m.at[pl.ds(r0 * N, RPW * N)])
    pltpu.sync_copy(l1_v, out_l1.at[pl.ds(r0 * L, RPW * L)])


def _sc_topk(s, e, g0rep, g1rep):
    mesh = plsc.VectorSubcoreMesh(core_axis_name="c", subcore_axis_name="s")
    f = pl.kernel(
        _sc_body,
        mesh=mesh,
        out_type=[
            jax.ShapeDtypeStruct((B * N,), jnp.float32),
            jax.ShapeDtypeStruct((B * L,), jnp.float32),
        ],
        scratch_types=[
            pltpu.VMEM((RPW * N,), jnp.float32),
            pltpu.VMEM((RPW * N,), jnp.float32),
            pltpu.VMEM((RPW * L,), jnp.float32),
            pltpu.VMEM((RPW * L,), jnp.float32),
            pltpu.VMEM((RPW * N,), jnp.float32),
            pltpu.VMEM((RPW * NG * L,), jnp.float32),
            pltpu.VMEM((RPW * N,), jnp.float32),
            pltpu.VMEM((RPW * L,), jnp.float32),
        ],
    )
    return f(s.reshape(-1), e.reshape(-1),
             g0rep.reshape(-1), g1rep.reshape(-1))


def _b1_body(s_ref, e_ref, sc_ref, gt_ref, base_ref, delta_ref):
    s = s_ref[:, :]
    e = e_ref[:, :]
    g0 = gt_ref[:, 0:1]
    g1 = gt_ref[:, 1:2]

    inter = jnp.clip(jnp.minimum(e, g1) - jnp.maximum(s, g0), 0.0)
    union = (e - s) + (g1 - g0) - inter
    enclose = jnp.maximum(e, g1) - jnp.minimum(s, g0)
    giou = inter / (union + EPS) - (enclose - union) / (enclose + EPS)

    t = jnp.where(giou < IOU_CUTOFF, 0.0, giou)
    l = sc_ref[:, :]
    ft = _focal_terms(l, t)
    base_ref[0, 0] = jnp.sum(ft)
    delta_ref[:, :] = _focal_terms(l, 1.0) - ft


def _b2_body(m_ref, d_ref, l1_ref, base_ref, out_ref):
    corr = jnp.sum(m_ref[:, :] * d_ref[:, :])
    l1_sum = jnp.sum(l1_ref[:, :])
    out_ref[0, 0] = ((base_ref[0, 0] + corr) / (B * N)
                     + l1_sum / (B * TOPK * 2))


def kernel(proposal, score, gt):
    s = proposal[:, :, 0]
    e = proposal[:, :, 1]
    g0rep = jnp.broadcast_to(gt[:, 0:1], (B, L))
    g1rep = jnp.broadcast_to(gt[:, 1:2], (B, L))

    base, delta = pl.pallas_call(
        _b1_body,
        out_shape=[
            jax.ShapeDtypeStruct((1, 1), jnp.float32),
            jax.ShapeDtypeStruct((B, N), jnp.float32),
        ],
        out_specs=[
            pl.BlockSpec(memory_space=pltpu.SMEM),
            pl.BlockSpec(memory_space=pltpu.VMEM),
        ],
    )(s, e, score, gt)

    mask, l1rows = _sc_topk(s, e, g0rep, g1rep)

    out = pl.pallas_call(
        _b2_body,
        out_shape=jax.ShapeDtypeStruct((1, 1), jnp.float32),
        out_specs=pl.BlockSpec(memory_space=pltpu.SMEM),
    )(mask.reshape(B, N), delta, l1rows.reshape(B, L), base)
    return out[0, 0]
